# baseline (speedup 1.0000x reference)
import functools

import jax
import jax.numpy as jnp
from jax import lax
from jax.experimental import pallas as pl
from jax.experimental.pallas import tpu as pltpu
from jax.experimental.pallas import tpu_sc as plsc

QUEUE = 16384
BATCH = 4096
NC, NS, L = 2, 16, 16
NW = NC * NS
NVREG = BATCH // L
G = 7 * 7 * 8
QT = QUEUE // 128
PT = BATCH // 128
NQ = 8
QQ = QT // NQ
CAP = 4096 + NQ * L
NG_PER = (G + NW - 1) // NW


def _sc_body(batch_hbm, idx_hbm, feat_hbm, out_hbm,
             idx_v, winner_v, pos_v, dst_v, blk0_v, blk1_v, blk2_v, bfb_v,
             lsem0, lsem1, lsem2, ssem0, ssem1, ssem2, bfsem):
    wid = lax.axis_index("s") * NC + lax.axis_index("c")
    iota = lax.iota(jnp.int32, L)
    zero = jnp.zeros((L,), jnp.int32)

    pltpu.sync_copy(idx_hbm, idx_v)

    def scan1(g, carry):
        x = idx_v[pl.ds(g * L, L)]
        posv = jnp.full((L,), g * L, jnp.int32) + iota
        keep = posv >= 0
        for s in range(1, L):
            sh = jnp.take_along_axis(x, jnp.minimum(iota + s, L - 1), axis=0)
            dup = (sh == x) & (iota < (L - s))
            keep = keep & (~dup)
        plsc.store_scatter(winner_v, [x], posv, mask=keep)
        return carry

    lax.fori_loop(0, NVREG, scan1, 0)

    def scanc(g, offs):
        x = idx_v[pl.ds(g * L, L)]
        posv = jnp.full((L,), g * L, jnp.int32) + iota
        w = plsc.load_gather(winner_v, [x])
        m = w == posv
        octv = jnp.right_shift(x, 11)
        return tuple(offs[e] + jnp.sum((m & (octv == e)).astype(jnp.int32))
                     for e in range(NQ))

    z = jnp.int32(0)
    cnts = lax.fori_loop(0, NVREG, scanc, (z,) * NQ)

    def _ceil16(c):
        return lax.div(c + jnp.int32(L - 1), jnp.int32(L))

    nv = [_ceil16(c) for c in cnts]
    base = [z]
    for e in range(1, NQ):
        base.append(base[e - 1] + nv[e - 1] * L)

    def scan2(g, offs):
        x = idx_v[pl.ds(g * L, L)]
        posv = jnp.full((L,), g * L, jnp.int32) + iota
        w = plsc.load_gather(winner_v, [x])
        m = w == posv
        octv = jnp.right_shift(x, 11)
        new_offs = []
        for e in range(NQ):
            me = m & (octv == e)
            ce = lax.cumsum(me.astype(jnp.int32), axis=0)
            re = jnp.full((L,), base[e] + offs[e], jnp.int32) + ce - 1
            plsc.store_scatter(pos_v, [re], posv, mask=me)
            plsc.store_scatter(dst_v, [re], x, mask=me)
            new_offs.append(offs[e] + jnp.sum(me.astype(jnp.int32)))
        return tuple(new_offs)

    lax.fori_loop(0, NVREG, scan2, (z,) * NQ)

    for e in range(NQ):
        rem = lax.rem(cnts[e], jnp.int32(L))

        @pl.when(rem != 0)
        def _p(e=e, rem=rem):
            bvec = jnp.full((L,), base[e], jnp.int32)
            p0 = plsc.load_gather(pos_v, [bvec])
            d0 = plsc.load_gather(dst_v, [bvec])
            lo = base[e] + cnts[e] - rem
            msk = iota < rem
            pos_v[pl.ds(lo, L)] = jnp.where(msk, pos_v[pl.ds(lo, L)], p0)
            dst_v[pl.ds(lo, L)] = jnp.where(msk, dst_v[pl.ds(lo, L)], d0)

    def _patch(h, blk):
        def pbody(j, carry):
            o = base[h] + j * L
            pos = pos_v[pl.ds(o, L)]
            dst = dst_v[pl.ds(o, L)]
            pt = jnp.right_shift(pos, 7)
            pi = jnp.bitwise_and(pos, 127)
            dtl = jnp.right_shift(dst, 7) - h * QQ
            di = jnp.bitwise_and(dst, 127)
            for s in range(8):
                fs = jnp.full((L,), s, jnp.int32)
                val = plsc.load_gather(bfb_v, [zero, pt, fs, pi])
                plsc.store_scatter(blk, [zero, dtl, fs, di], val)
            return carry

        lax.fori_loop(0, nv[h], pbody, 0)

    blks = (blk0_v, blk1_v, blk2_v)
    lsems = (lsem0, lsem1, lsem2)
    ssems = (ssem0, ssem1, ssem2)

    def _ld(g, h):
        return pltpu.async_copy(
            feat_hbm.at[pl.ds(g, 1), pl.ds(h * QQ, QQ)], blks[h % 3],
            lsems[h % 3])

    def _st(g, h):
        return pltpu.async_copy(
            blks[h % 3], out_hbm.at[pl.ds(g, 1), pl.ds(h * QQ, QQ)],
            ssems[h % 3])

    def kbody(k, carry):
        g = jnp.minimum(wid + k * NW, G - 1)
        bfh = pltpu.async_copy(batch_hbm.at[pl.ds(g, 1)], bfb_v, bfsem)
        ld = {0: _ld(g, 0), 1: _ld(g, 1), 2: _ld(g, 2)}
        st = {}
        bfh.wait()
        for e in range(NQ):
            b = e % 3
            ld[e].wait()
            if e >= 2 and e + 1 < NQ:
                st[e - 2].wait()
                ld[e + 1] = _ld(g, e + 1)
            _patch(e, blks[b])
            st[e] = _st(g, e)
        st[NQ - 3].wait()
        st[NQ - 2].wait()
        st[NQ - 1].wait()
        return carry

    lax.fori_loop(0, NG_PER, kbody, 0)


_sc_call = functools.partial(
    pl.kernel,
    out_type=jax.ShapeDtypeStruct((G, QT, 8, 128), jnp.float32),
    mesh=plsc.VectorSubcoreMesh(core_axis_name="c", subcore_axis_name="s"),
    compiler_params=pltpu.CompilerParams(needs_layout_passes=False),
    scratch_types=[
        pltpu.VMEM((BATCH,), jnp.int32),
        pltpu.VMEM((QUEUE,), jnp.int32),
        pltpu.VMEM((CAP,), jnp.int32),
        pltpu.VMEM((CAP,), jnp.int32),
        pltpu.VMEM((1, QQ, 8, 128), jnp.float32),
        pltpu.VMEM((1, QQ, 8, 128), jnp.float32),
        pltpu.VMEM((1, QQ, 8, 128), jnp.float32),
        pltpu.VMEM((1, PT, 8, 128), jnp.float32),
        pltpu.SemaphoreType.DMA,
        pltpu.SemaphoreType.DMA,
        pltpu.SemaphoreType.DMA,
        pltpu.SemaphoreType.DMA,
        pltpu.SemaphoreType.DMA,
        pltpu.SemaphoreType.DMA,
        pltpu.SemaphoreType.DMA,
    ],
)(_sc_body)


def kernel(batch_features, batch_indices, features):
    bf = (batch_features.transpose(2, 3, 1, 0)
          .reshape(7, 7, 8, 8, PT, 128).transpose(0, 1, 2, 4, 3, 5)
          .reshape(G, PT, 8, 128))
    ft = (features.transpose(2, 3, 1, 0)
          .reshape(7, 7, 8, 8, QT, 128).transpose(0, 1, 2, 4, 3, 5)
          .reshape(G, QT, 8, 128))
    out = _sc_call(bf, batch_indices, ft)
    return (out.reshape(7, 7, 8, QT, 8, 128).transpose(0, 1, 2, 4, 3, 5)
            .reshape(7, 7, 64, QUEUE).transpose(3, 2, 0, 1))


# speedup vs baseline: 1.1853x; 1.1853x over previous
import functools

import jax
import jax.numpy as jnp
from jax import lax
from jax.experimental import pallas as pl
from jax.experimental.pallas import tpu as pltpu
from jax.experimental.pallas import tpu_sc as plsc

QUEUE = 16384
BATCH = 4096
NC, NS, L = 2, 16, 16
NW = NC * NS
NVREG = BATCH // L
G = 7 * 7 * 8
QT = QUEUE // 128
PT = BATCH // 128
NQ = 8
QQ = QT // NQ
CAP = 4096 + NQ * L
NG_PER = (G + NW - 1) // NW


def _sc_body(batch_hbm, idx_hbm, feat_hbm, out_hbm,
             idx_v, winner_v, pos_v, dst_v,
             blk0_v, blk1_v, blk2_v, blk3_v, bfb_v,
             lsem0, lsem1, lsem2, lsem3, ssem0, ssem1, ssem2, ssem3, bfsem):
    wid = lax.axis_index("s") * NC + lax.axis_index("c")
    iota = lax.iota(jnp.int32, L)
    zero = jnp.zeros((L,), jnp.int32)

    pltpu.sync_copy(idx_hbm, idx_v)

    def scan1(g, carry):
        x = idx_v[pl.ds(g * L, L)]
        posv = jnp.full((L,), g * L, jnp.int32) + iota
        keep = posv >= 0
        for s in range(1, L):
            sh = jnp.take_along_axis(x, jnp.minimum(iota + s, L - 1), axis=0)
            dup = (sh == x) & (iota < (L - s))
            keep = keep & (~dup)
        plsc.store_scatter(winner_v, [x], posv, mask=keep)
        return carry

    lax.fori_loop(0, NVREG, scan1, 0)

    def scanc(g, offs):
        x = idx_v[pl.ds(g * L, L)]
        posv = jnp.full((L,), g * L, jnp.int32) + iota
        w = plsc.load_gather(winner_v, [x])
        m = w == posv
        octv = jnp.right_shift(x, 11)
        return tuple(offs[e] + jnp.sum((m & (octv == e)).astype(jnp.int32))
                     for e in range(NQ))

    z = jnp.int32(0)
    cnts = lax.fori_loop(0, NVREG, scanc, (z,) * NQ)

    def _ceil16(c):
        return lax.div(c + jnp.int32(L - 1), jnp.int32(L))

    nv = [_ceil16(c) for c in cnts]
    base = [z]
    for e in range(1, NQ):
        base.append(base[e - 1] + nv[e - 1] * L)

    def scan2(g, offs):
        x = idx_v[pl.ds(g * L, L)]
        posv = jnp.full((L,), g * L, jnp.int32) + iota
        w = plsc.load_gather(winner_v, [x])
        m = w == posv
        octv = jnp.right_shift(x, 11)
        new_offs = []
        for e in range(NQ):
            me = m & (octv == e)
            ce = lax.cumsum(me.astype(jnp.int32), axis=0)
            re = jnp.full((L,), base[e] + offs[e], jnp.int32) + ce - 1
            plsc.store_scatter(pos_v, [re], posv, mask=me)
            plsc.store_scatter(dst_v, [re], x, mask=me)
            new_offs.append(offs[e] + jnp.sum(me.astype(jnp.int32)))
        return tuple(new_offs)

    lax.fori_loop(0, NVREG, scan2, (z,) * NQ)

    for e in range(NQ):
        rem = lax.rem(cnts[e], jnp.int32(L))

        @pl.when(rem != 0)
        def _p(e=e, rem=rem):
            bvec = jnp.full((L,), base[e], jnp.int32)
            p0 = plsc.load_gather(pos_v, [bvec])
            d0 = plsc.load_gather(dst_v, [bvec])
            lo = base[e] + cnts[e] - rem
            msk = iota < rem
            pos_v[pl.ds(lo, L)] = jnp.where(msk, pos_v[pl.ds(lo, L)], p0)
            dst_v[pl.ds(lo, L)] = jnp.where(msk, dst_v[pl.ds(lo, L)], d0)

    def _patch(h, blk):
        def pbody(j, carry):
            o = base[h] + j * L
            pos = pos_v[pl.ds(o, L)]
            dst = dst_v[pl.ds(o, L)]
            pt = jnp.right_shift(pos, 7)
            pi = jnp.bitwise_and(pos, 127)
            dtl = jnp.right_shift(dst, 7) - h * QQ
            di = jnp.bitwise_and(dst, 127)
            for s in range(8):
                fs = jnp.full((L,), s, jnp.int32)
                val = plsc.load_gather(bfb_v, [zero, pt, fs, pi])
                plsc.store_scatter(blk, [zero, dtl, fs, di], val)
            return carry

        lax.fori_loop(0, nv[h], pbody, 0)

    blks = (blk0_v, blk1_v, blk2_v, blk3_v)
    lsems = (lsem0, lsem1, lsem2, lsem3)
    ssems = (ssem0, ssem1, ssem2, ssem3)

    def _ld(g, h):
        return pltpu.async_copy(
            feat_hbm.at[pl.ds(g, 1), pl.ds(h * QQ, QQ)], blks[h % 4],
            lsems[h % 4])

    def _st(g, h):
        return pltpu.async_copy(
            blks[h % 4], out_hbm.at[pl.ds(g, 1), pl.ds(h * QQ, QQ)],
            ssems[h % 4])

    def _round(g):
        bfh = pltpu.async_copy(batch_hbm.at[pl.ds(g, 1)], bfb_v, bfsem)
        ld = {0: _ld(g, 0), 1: _ld(g, 1)}
        st = {}
        bfh.wait()
        for e in range(NQ):
            b = e % 4
            if e + 2 < NQ:
                if e >= 2:
                    st[e - 2].wait()
                ld[e + 2] = _ld(g, e + 2)
            ld[e].wait()
            _patch(e, blks[b])
            st[e] = _st(g, e)
        for e in range(NQ - 4, NQ):
            st[e].wait()

    def kbody(k, carry):
        _round(wid + k * NW)
        return carry

    lax.fori_loop(0, NG_PER - 1, kbody, 0)

    @pl.when(wid + (NG_PER - 1) * NW < G)
    def _tail():
        _round(wid + (NG_PER - 1) * NW)


_sc_call = functools.partial(
    pl.kernel,
    out_type=jax.ShapeDtypeStruct((G, QT, 8, 128), jnp.float32),
    mesh=plsc.VectorSubcoreMesh(core_axis_name="c", subcore_axis_name="s"),
    compiler_params=pltpu.CompilerParams(needs_layout_passes=False),
    scratch_types=[
        pltpu.VMEM((BATCH,), jnp.int32),
        pltpu.VMEM((QUEUE,), jnp.int32),
        pltpu.VMEM((CAP,), jnp.int32),
        pltpu.VMEM((CAP,), jnp.int32),
        pltpu.VMEM((1, QQ, 8, 128), jnp.float32),
        pltpu.VMEM((1, QQ, 8, 128), jnp.float32),
        pltpu.VMEM((1, QQ, 8, 128), jnp.float32),
        pltpu.VMEM((1, QQ, 8, 128), jnp.float32),
        pltpu.VMEM((1, PT, 8, 128), jnp.float32),
        pltpu.SemaphoreType.DMA,
        pltpu.SemaphoreType.DMA,
        pltpu.SemaphoreType.DMA,
        pltpu.SemaphoreType.DMA,
        pltpu.SemaphoreType.DMA,
        pltpu.SemaphoreType.DMA,
        pltpu.SemaphoreType.DMA,
        pltpu.SemaphoreType.DMA,
        pltpu.SemaphoreType.DMA,
    ],
)(_sc_body)


def kernel(batch_features, batch_indices, features):
    bf = (batch_features.transpose(2, 3, 1, 0)
          .reshape(7, 7, 8, 8, PT, 128).transpose(0, 1, 2, 4, 3, 5)
          .reshape(G, PT, 8, 128))
    ft = (features.transpose(2, 3, 1, 0)
          .reshape(7, 7, 8, 8, QT, 128).transpose(0, 1, 2, 4, 3, 5)
          .reshape(G, QT, 8, 128))
    out = _sc_call(bf, batch_indices, ft)
    return (out.reshape(7, 7, 8, QT, 8, 128).transpose(0, 1, 2, 4, 3, 5)
            .reshape(7, 7, 64, QUEUE).transpose(3, 2, 0, 1))


# speedup vs baseline: 1.1950x; 1.0082x over previous
import functools

import jax
import jax.numpy as jnp
from jax import lax
from jax.experimental import pallas as pl
from jax.experimental.pallas import tpu as pltpu
from jax.experimental.pallas import tpu_sc as plsc

QUEUE = 16384
BATCH = 4096
NC, NS, L = 2, 16, 16
NW = NC * NS
NVREG = BATCH // L
G = 7 * 7 * 8
QT = QUEUE // 128
PT = BATCH // 128
NQ = 8
QQ = QT // NQ
CAP = 4096 + NQ * L
NG_PER = (G + NW - 1) // NW


def _sc_body(batch_hbm, idx_hbm, feat_hbm, out_hbm,
             idx_v, winner_v, pos_v, dst_v,
             blk0_v, blk1_v, blk2_v, blk3_v, bfb_v,
             lsem0, lsem1, lsem2, lsem3, ssem0, ssem1, ssem2, ssem3, bfsem):
    wid = lax.axis_index("s") * NC + lax.axis_index("c")
    iota = lax.iota(jnp.int32, L)
    zero = jnp.zeros((L,), jnp.int32)

    pltpu.sync_copy(idx_hbm, idx_v)

    _bfh0 = pltpu.async_copy(batch_hbm.at[pl.ds(wid, 1)], bfb_v, bfsem)
    _ld00 = pltpu.async_copy(
        feat_hbm.at[pl.ds(wid, 1), pl.ds(0, QQ)], blk0_v, lsem0)
    _ld01 = pltpu.async_copy(
        feat_hbm.at[pl.ds(wid, 1), pl.ds(QQ, QQ)], blk1_v, lsem1)

    def scan1(g, carry):
        x = idx_v[pl.ds(g * L, L)]
        posv = jnp.full((L,), g * L, jnp.int32) + iota
        keep = posv >= 0
        for s in range(1, L):
            sh = jnp.take_along_axis(x, jnp.minimum(iota + s, L - 1), axis=0)
            dup = (sh == x) & (iota < (L - s))
            keep = keep & (~dup)
        plsc.store_scatter(winner_v, [x], posv, mask=keep)
        return carry

    lax.fori_loop(0, NVREG, scan1, 0)

    def scanc(g, offs):
        x = idx_v[pl.ds(g * L, L)]
        posv = jnp.full((L,), g * L, jnp.int32) + iota
        w = plsc.load_gather(winner_v, [x])
        m = w == posv
        octv = jnp.right_shift(x, 11)
        return tuple(offs[e] + jnp.sum((m & (octv == e)).astype(jnp.int32))
                     for e in range(NQ))

    z = jnp.int32(0)
    cnts = lax.fori_loop(0, NVREG, scanc, (z,) * NQ)

    def _ceil16(c):
        return lax.div(c + jnp.int32(L - 1), jnp.int32(L))

    nv = [_ceil16(c) for c in cnts]
    base = [z]
    for e in range(1, NQ):
        base.append(base[e - 1] + nv[e - 1] * L)

    def scan2(g, offs):
        x = idx_v[pl.ds(g * L, L)]
        posv = jnp.full((L,), g * L, jnp.int32) + iota
        w = plsc.load_gather(winner_v, [x])
        m = w == posv
        octv = jnp.right_shift(x, 11)
        new_offs = []
        for e in range(NQ):
            me = m & (octv == e)
            ce = lax.cumsum(me.astype(jnp.int32), axis=0)
            re = jnp.full((L,), base[e] + offs[e], jnp.int32) + ce - 1
            plsc.store_scatter(pos_v, [re], posv, mask=me)
            plsc.store_scatter(dst_v, [re], x, mask=me)
            new_offs.append(offs[e] + jnp.sum(me.astype(jnp.int32)))
        return tuple(new_offs)

    lax.fori_loop(0, NVREG, scan2, (z,) * NQ)

    for e in range(NQ):
        rem = lax.rem(cnts[e], jnp.int32(L))

        @pl.when(rem != 0)
        def _p(e=e, rem=rem):
            bvec = jnp.full((L,), base[e], jnp.int32)
            p0 = plsc.load_gather(pos_v, [bvec])
            d0 = plsc.load_gather(dst_v, [bvec])
            lo = base[e] + cnts[e] - rem
            msk = iota < rem
            pos_v[pl.ds(lo, L)] = jnp.where(msk, pos_v[pl.ds(lo, L)], p0)
            dst_v[pl.ds(lo, L)] = jnp.where(msk, dst_v[pl.ds(lo, L)], d0)

    def _patch(h, blk):
        def pbody(j, carry):
            o = base[h] + j * L
            pos = pos_v[pl.ds(o, L)]
            dst = dst_v[pl.ds(o, L)]
            pt = jnp.right_shift(pos, 7)
            pi = jnp.bitwise_and(pos, 127)
            dtl = jnp.right_shift(dst, 7) - h * QQ
            di = jnp.bitwise_and(dst, 127)
            for s in range(8):
                fs = jnp.full((L,), s, jnp.int32)
                val = plsc.load_gather(bfb_v, [zero, pt, fs, pi])
                plsc.store_scatter(blk, [zero, dtl, fs, di], val)
            return carry

        lax.fori_loop(0, nv[h], pbody, 0)

    blks = (blk0_v, blk1_v, blk2_v, blk3_v)
    lsems = (lsem0, lsem1, lsem2, lsem3)
    ssems = (ssem0, ssem1, ssem2, ssem3)

    def _ld(g, h):
        return pltpu.async_copy(
            feat_hbm.at[pl.ds(g, 1), pl.ds(h * QQ, QQ)], blks[h % 4],
            lsems[h % 4])

    def _st(g, h):
        return pltpu.async_copy(
            blks[h % 4], out_hbm.at[pl.ds(g, 1), pl.ds(h * QQ, QQ)],
            ssems[h % 4])

    def _round(g, pre=None):
        if pre is None:
            bfh = pltpu.async_copy(batch_hbm.at[pl.ds(g, 1)], bfb_v, bfsem)
            ld = {0: _ld(g, 0), 1: _ld(g, 1)}
        else:
            bfh, ld = pre
        st = {}
        bfh.wait()
        for e in range(NQ):
            b = e % 4
            if e + 2 < NQ:
                if e >= 2:
                    st[e - 2].wait()
                ld[e + 2] = _ld(g, e + 2)
            ld[e].wait()
            _patch(e, blks[b])
            st[e] = _st(g, e)
        for e in range(NQ - 4, NQ):
            st[e].wait()

    _round(wid, pre=(_bfh0, {0: _ld00, 1: _ld01}))

    def kbody(k, carry):
        _round(wid + k * NW)
        return carry

    lax.fori_loop(1, NG_PER - 1, kbody, 0)

    @pl.when(wid + (NG_PER - 1) * NW < G)
    def _tail():
        _round(wid + (NG_PER - 1) * NW)


_sc_call = functools.partial(
    pl.kernel,
    out_type=jax.ShapeDtypeStruct((G, QT, 8, 128), jnp.float32),
    mesh=plsc.VectorSubcoreMesh(core_axis_name="c", subcore_axis_name="s"),
    compiler_params=pltpu.CompilerParams(needs_layout_passes=False),
    scratch_types=[
        pltpu.VMEM((BATCH,), jnp.int32),
        pltpu.VMEM((QUEUE,), jnp.int32),
        pltpu.VMEM((CAP,), jnp.int32),
        pltpu.VMEM((CAP,), jnp.int32),
        pltpu.VMEM((1, QQ, 8, 128), jnp.float32),
        pltpu.VMEM((1, QQ, 8, 128), jnp.float32),
        pltpu.VMEM((1, QQ, 8, 128), jnp.float32),
        pltpu.VMEM((1, QQ, 8, 128), jnp.float32),
        pltpu.VMEM((1, PT, 8, 128), jnp.float32),
        pltpu.SemaphoreType.DMA,
        pltpu.SemaphoreType.DMA,
        pltpu.SemaphoreType.DMA,
        pltpu.SemaphoreType.DMA,
        pltpu.SemaphoreType.DMA,
        pltpu.SemaphoreType.DMA,
        pltpu.SemaphoreType.DMA,
        pltpu.SemaphoreType.DMA,
        pltpu.SemaphoreType.DMA,
    ],
)(_sc_body)


def kernel(batch_features, batch_indices, features):
    bf = (batch_features.transpose(2, 3, 1, 0)
          .reshape(7, 7, 8, 8, PT, 128).transpose(0, 1, 2, 4, 3, 5)
          .reshape(G, PT, 8, 128))
    ft = (features.transpose(2, 3, 1, 0)
          .reshape(7, 7, 8, 8, QT, 128).transpose(0, 1, 2, 4, 3, 5)
          .reshape(G, QT, 8, 128))
    out = _sc_call(bf, batch_indices, ft)
    return (out.reshape(7, 7, 8, QT, 8, 128).transpose(0, 1, 2, 4, 3, 5)
            .reshape(7, 7, 64, QUEUE).transpose(3, 2, 0, 1))
